# Initial kernel scaffold; baseline (speedup 1.0000x reference)
import jax
import jax.numpy as jnp
from jax.experimental import pallas as pl


def kernel(items, H, pair_r, pair_c, emb, W2_1, W3_1, a_1, a2_1, wc_1, W_2, W2_2, W3_2, a_2, a2_2, wc_2, lin_W, lin_b):
    raise NotImplementedError("write your pallas kernel here")


# speedup vs baseline: 3.9909x; 3.9909x over previous
import functools

import jax
import jax.numpy as jnp
from jax.experimental import pallas as pl
from jax.experimental.pallas import tpu as pltpu

N = 4096
M = 2048
D = 256
NEG = float(-9e15)

_EMB_WAYS = 8


def _embed_kernel(items_ref, *refs):
    out_ref = refs[-1]
    for j in range(_EMB_WAYS):
        out_ref[j, :] = refs[j][0, 0, :]


def _gather_rows(emb, items):
    n = items.shape[0]
    d = emb.shape[1]
    emb3 = emb.reshape(emb.shape[0], 1, d)

    def emb_map(j, i, items_ref):
        return (items_ref[i * _EMB_WAYS + j], 0, 0)

    emb_specs = [
        pl.BlockSpec((1, 1, d), functools.partial(emb_map, j)) for j in range(_EMB_WAYS)
    ]
    return pl.pallas_call(
        _embed_kernel,
        grid_spec=pltpu.PrefetchScalarGridSpec(
            num_scalar_prefetch=1,
            grid=(n // _EMB_WAYS,),
            in_specs=emb_specs,
            out_specs=pl.BlockSpec((_EMB_WAYS, d), lambda i, items_ref: (i, 0)),
        ),
        out_shape=jax.ShapeDtypeStruct((n, d), emb.dtype),
    )(items, *([emb3] * _EMB_WAYS))


def _prep_kernel(x_ref, W2_ref, W3_ref, a_ref, a2_ref, wc_ref,
                 ls_ref, t_ref, v3T_ref):
    a_top = a_ref[:D, :]
    a_bot = a_ref[D:, :]
    a2_top = a2_ref[:D, :]
    a2_bot = a2_ref[D:, :]
    c0 = jnp.sum(wc_ref[:, :] * a_top)
    v_s = jnp.dot(W2_ref[:, :], a_bot, preferred_element_type=jnp.float32)
    v_t = jnp.dot(W2_ref[:, :], a2_top, preferred_element_type=jnp.float32)
    s = jnp.dot(x_ref[:, :], v_s, preferred_element_type=jnp.float32) + c0
    ls_ref[:, :] = jnp.where(s >= 0, s, 0.2 * s)
    t_ref[:, :] = jnp.dot(x_ref[:, :], v_t, preferred_element_type=jnp.float32)
    v3T_ref[:, :] = jax.lax.dot_general(
        a2_bot, W3_ref[:, :], (((0,), (1,)), ((), ())),
        preferred_element_type=jnp.float32)


def _prep(x, W2, W3, a, a2, wc):
    wc2 = wc.reshape(D, 1)
    out_shapes = (
        jax.ShapeDtypeStruct((N, 1), jnp.float32),
        jax.ShapeDtypeStruct((N, 1), jnp.float32),
        jax.ShapeDtypeStruct((1, D), jnp.float32),
    )
    return pl.pallas_call(
        _prep_kernel,
        out_shape=out_shapes,
    )(x, W2, W3, a, a2, wc2)


def _prep2_kernel(x_ref, W2_ref, W3_ref, a_ref, a2_ref, wc_ref, W_ref,
                  ls_ref, t_ref, v3T_ref, xw_ref):
    _prep_kernel(x_ref, W2_ref, W3_ref, a_ref, a2_ref, wc_ref,
                 ls_ref, t_ref, v3T_ref)
    xw_ref[:, :] = jnp.dot(x_ref[:, :], W_ref[:, :],
                           preferred_element_type=jnp.float32)


def _prep2(x, W2, W3, a, a2, wc, W):
    wc2 = wc.reshape(D, 1)
    out_shapes = (
        jax.ShapeDtypeStruct((N, 1), jnp.float32),
        jax.ShapeDtypeStruct((N, 1), jnp.float32),
        jax.ShapeDtypeStruct((1, D), jnp.float32),
        jax.ShapeDtypeStruct((N, D), jnp.float32),
    )
    return pl.pallas_call(
        _prep2_kernel,
        out_shape=out_shapes,
    )(x, W2, W3, a, a2, wc2, W)


_BM = 256


def _edge_u_kernel(H_ref, ls_ref, xa_ref, v3_ref, edge_ref, u_ref):
    h = H_ref[:, :]
    e = jnp.where(h > 0, ls_ref[:, :], NEG)
    mx = jnp.max(e, axis=0, keepdims=True)
    p = jnp.exp(e - mx)
    att = p / jnp.sum(p, axis=0, keepdims=True)
    edge = jax.lax.dot_general(
        att, xa_ref[:, :], (((0,), (0,)), ((), ())),
        preferred_element_type=jnp.float32)
    edge_ref[:, :] = edge
    u_ref[:, :] = jax.lax.dot_general(
        edge, v3_ref[:, :], (((1,), (1,)), ((), ())),
        preferred_element_type=jnp.float32)


def _edge_phase(H, ls, xa, v3T):
    grid = (M // _BM,)
    out_shapes = (
        jax.ShapeDtypeStruct((M, D), jnp.float32),
        jax.ShapeDtypeStruct((M, 1), jnp.float32),
    )
    return pl.pallas_call(
        _edge_u_kernel,
        grid=grid,
        in_specs=[
            pl.BlockSpec((N, _BM), lambda i: (0, i)),
            pl.BlockSpec((N, 1), lambda i: (0, 0)),
            pl.BlockSpec((N, D), lambda i: (0, 0)),
            pl.BlockSpec((1, D), lambda i: (0, 0)),
        ],
        out_specs=(
            pl.BlockSpec((_BM, D), lambda i: (i, 0)),
            pl.BlockSpec((_BM, 1), lambda i: (i, 0)),
        ),
        out_shape=out_shapes,
    )(H, ls, xa, v3T)


_BN = 512


def _node_kernel(H_ref, t_ref, u_ref, edge_ref, out_ref):
    z = t_ref[:, :] + u_ref[:, :]
    z = jnp.where(z >= 0, z, 0.2 * z)
    e2 = jnp.where(H_ref[:, :] > 0, z, NEG)
    mx = jnp.max(e2, axis=1, keepdims=True)
    p = jnp.exp(e2 - mx)
    att2 = p / jnp.sum(p, axis=1, keepdims=True)
    nd = jnp.dot(att2, edge_ref[:, :], preferred_element_type=jnp.float32)
    out_ref[:, :] = jnp.where(nd > 0, nd, jnp.exp(nd) - 1.0)


def _node_phase(H, t, u_row, edge):
    grid = (N // _BN,)
    return pl.pallas_call(
        _node_kernel,
        grid=grid,
        in_specs=[
            pl.BlockSpec((_BN, M), lambda i: (i, 0)),
            pl.BlockSpec((_BN, 1), lambda i: (i, 0)),
            pl.BlockSpec((1, M), lambda i: (0, 0)),
            pl.BlockSpec((M, D), lambda i: (0, 0)),
        ],
        out_specs=pl.BlockSpec((_BN, D), lambda i: (i, 0)),
        out_shape=jax.ShapeDtypeStruct((N, D), jnp.float32),
    )(H, t, u_row, edge)


def _node_lin_kernel(H_ref, t_ref, u_ref, edge_ref, linW_ref, linb_ref, out_ref):
    z = t_ref[:, :] + u_ref[:, :]
    z = jnp.where(z >= 0, z, 0.2 * z)
    e2 = jnp.where(H_ref[:, :] > 0, z, NEG)
    mx = jnp.max(e2, axis=1, keepdims=True)
    p = jnp.exp(e2 - mx)
    att2 = p / jnp.sum(p, axis=1, keepdims=True)
    nd = jnp.dot(att2, edge_ref[:, :], preferred_element_type=jnp.float32)
    h = jnp.where(nd > 0, nd, jnp.exp(nd) - 1.0)
    out_ref[:, :] = jax.lax.dot_general(
        h, linW_ref[:, :], (((1,), (1,)), ((), ())),
        preferred_element_type=jnp.float32) + linb_ref[:, :]


def _node_phase_lin(H, t, u_row, edge, lin_W, lin_b):
    ncat = lin_W.shape[0]
    grid = (N // _BN,)
    return pl.pallas_call(
        _node_lin_kernel,
        grid=grid,
        in_specs=[
            pl.BlockSpec((_BN, M), lambda i: (i, 0)),
            pl.BlockSpec((_BN, 1), lambda i: (i, 0)),
            pl.BlockSpec((1, M), lambda i: (0, 0)),
            pl.BlockSpec((M, D), lambda i: (0, 0)),
            pl.BlockSpec((ncat, D), lambda i: (0, 0)),
            pl.BlockSpec((1, ncat), lambda i: (0, 0)),
        ],
        out_specs=pl.BlockSpec((_BN, ncat), lambda i: (i, 0)),
        out_shape=jax.ShapeDtypeStruct((N, ncat), jnp.float32),
    )(H, t, u_row, edge, lin_W, lin_b.reshape(1, ncat))


def kernel(items, H, pair_r, pair_c, emb, W2_1, W3_1, a_1, a2_1, wc_1,
           W_2, W2_2, W3_2, a_2, a2_2, wc_2, lin_W, lin_b):
    x0 = _gather_rows(emb, items)

    ls1, t1, v3T1 = _prep(x0, W2_1, W3_1, a_1, a2_1, wc_1)
    edge1, u1 = _edge_phase(H, ls1, x0, v3T1)
    h1 = _node_phase(H, t1, u1.reshape(1, M), edge1)

    ls2, t2, v3T2, xw = _prep2(h1, W2_2, W3_2, a_2, a2_2, wc_2, W_2)
    edge2, u2 = _edge_phase(H, ls2, xw, v3T2)
    out = _node_phase_lin(H, t2, u2.reshape(1, M), edge2, lin_W, lin_b)
    return out


# speedup vs baseline: 14.6101x; 3.6609x over previous
import functools

import jax
import jax.numpy as jnp
from jax.experimental import pallas as pl
from jax.experimental.pallas import tpu as pltpu
from jax.experimental.pallas import tpu_sc as plsc

N = 4096
M = 2048
D = 256
NEG = float(-9e15)

_EMB_WAYS = 8


def _embed_kernel(items_ref, *refs):
    out_ref = refs[-1]
    for j in range(_EMB_WAYS):
        out_ref[j, :] = refs[j][0, 0, :]


def _gather_rows(emb, items):
    n = items.shape[0]
    d = emb.shape[1]
    emb3 = emb.reshape(emb.shape[0], 1, d)

    def emb_map(j, i, items_ref):
        return (items_ref[i * _EMB_WAYS + j], 0, 0)

    emb_specs = [
        pl.BlockSpec((1, 1, d), functools.partial(emb_map, j)) for j in range(_EMB_WAYS)
    ]
    return pl.pallas_call(
        _embed_kernel,
        grid_spec=pltpu.PrefetchScalarGridSpec(
            num_scalar_prefetch=1,
            grid=(n // _EMB_WAYS,),
            in_specs=emb_specs,
            out_specs=pl.BlockSpec((_EMB_WAYS, d), lambda i, items_ref: (i, 0)),
        ),
        out_shape=jax.ShapeDtypeStruct((n, d), emb.dtype),
    )(items, *([emb3] * _EMB_WAYS))


def _gather_rows_sc(emb, items):
    info = plsc.get_sparse_core_info()
    nw = info.num_cores * info.num_subcores
    b = items.shape[0]
    d = emb.shape[1]
    b_per_w = b // nw
    mesh = plsc.VectorSubcoreMesh(core_axis_name="c", subcore_axis_name="s")

    @functools.partial(
        pl.kernel, mesh=mesh,
        out_type=jax.ShapeDtypeStruct((b, d), emb.dtype),
        scratch_types=[
            pltpu.VMEM((b_per_w,), jnp.int32),
            pltpu.VMEM((b_per_w, d), jnp.float32),
            pltpu.SemaphoreType.DMA,
        ],
    )
    def k(emb_hbm, items_hbm, out_hbm, idx_v, rows_v, sem):
        wid = jax.lax.axis_index("s") * info.num_cores + jax.lax.axis_index("c")
        base = wid * b_per_w
        pltpu.sync_copy(items_hbm.at[pl.ds(base, b_per_w)], idx_v)
        pltpu.async_copy(emb_hbm.at[idx_v], rows_v, sem).wait()
        pltpu.sync_copy(rows_v, out_hbm.at[pl.ds(base, b_per_w)])

    return k(emb, items)


def _prep_kernel(x_ref, W2_ref, W3_ref, a_ref, a2_ref, wc_ref,
                 ls_ref, t_ref, v3T_ref):
    a_top = a_ref[:D, :]
    a_bot = a_ref[D:, :]
    a2_top = a2_ref[:D, :]
    a2_bot = a2_ref[D:, :]
    c0 = jnp.sum(wc_ref[:, :] * a_top)
    v_s = jnp.dot(W2_ref[:, :], a_bot, preferred_element_type=jnp.float32)
    v_t = jnp.dot(W2_ref[:, :], a2_top, preferred_element_type=jnp.float32)
    s = jnp.dot(x_ref[:, :], v_s, preferred_element_type=jnp.float32) + c0
    ls_ref[:, :] = jnp.where(s >= 0, s, 0.2 * s)
    t_ref[:, :] = jnp.dot(x_ref[:, :], v_t, preferred_element_type=jnp.float32)
    v3T_ref[:, :] = jax.lax.dot_general(
        a2_bot, W3_ref[:, :], (((0,), (1,)), ((), ())),
        preferred_element_type=jnp.float32)


def _prep(x, W2, W3, a, a2, wc):
    wc2 = wc.reshape(D, 1)
    out_shapes = (
        jax.ShapeDtypeStruct((N, 1), jnp.float32),
        jax.ShapeDtypeStruct((N, 1), jnp.float32),
        jax.ShapeDtypeStruct((1, D), jnp.float32),
    )
    return pl.pallas_call(
        _prep_kernel,
        out_shape=out_shapes,
    )(x, W2, W3, a, a2, wc2)


def _prep2_kernel(x_ref, W2_ref, W3_ref, a_ref, a2_ref, wc_ref, W_ref,
                  ls_ref, t_ref, v3T_ref, xw_ref):
    _prep_kernel(x_ref, W2_ref, W3_ref, a_ref, a2_ref, wc_ref,
                 ls_ref, t_ref, v3T_ref)
    xw_ref[:, :] = jnp.dot(x_ref[:, :], W_ref[:, :],
                           preferred_element_type=jnp.float32)


def _prep2(x, W2, W3, a, a2, wc, W):
    wc2 = wc.reshape(D, 1)
    out_shapes = (
        jax.ShapeDtypeStruct((N, 1), jnp.float32),
        jax.ShapeDtypeStruct((N, 1), jnp.float32),
        jax.ShapeDtypeStruct((1, D), jnp.float32),
        jax.ShapeDtypeStruct((N, D), jnp.float32),
    )
    return pl.pallas_call(
        _prep2_kernel,
        out_shape=out_shapes,
    )(x, W2, W3, a, a2, wc2, W)


_BM = 256


def _edge_u_kernel(H_ref, ls_ref, xa_ref, v3_ref, edge_ref, u_ref):
    h = H_ref[:, :]
    e = jnp.where(h > 0, ls_ref[:, :], NEG)
    mx = jnp.max(e, axis=0, keepdims=True)
    p = jnp.exp(e - mx)
    att = p / jnp.sum(p, axis=0, keepdims=True)
    edge = jax.lax.dot_general(
        att, xa_ref[:, :], (((0,), (0,)), ((), ())),
        preferred_element_type=jnp.float32)
    edge_ref[:, :] = edge
    u_ref[:, :] = jax.lax.dot_general(
        edge, v3_ref[:, :], (((1,), (1,)), ((), ())),
        preferred_element_type=jnp.float32)


def _edge_phase(H, ls, xa, v3T):
    grid = (M // _BM,)
    out_shapes = (
        jax.ShapeDtypeStruct((M, D), jnp.float32),
        jax.ShapeDtypeStruct((M, 1), jnp.float32),
    )
    return pl.pallas_call(
        _edge_u_kernel,
        grid=grid,
        in_specs=[
            pl.BlockSpec((N, _BM), lambda i: (0, i)),
            pl.BlockSpec((N, 1), lambda i: (0, 0)),
            pl.BlockSpec((N, D), lambda i: (0, 0)),
            pl.BlockSpec((1, D), lambda i: (0, 0)),
        ],
        out_specs=(
            pl.BlockSpec((_BM, D), lambda i: (i, 0)),
            pl.BlockSpec((_BM, 1), lambda i: (i, 0)),
        ),
        out_shape=out_shapes,
    )(H, ls, xa, v3T)


_BN = 512


def _node_kernel(H_ref, t_ref, u_ref, edge_ref, out_ref):
    z = t_ref[:, :] + u_ref[:, :]
    z = jnp.where(z >= 0, z, 0.2 * z)
    e2 = jnp.where(H_ref[:, :] > 0, z, NEG)
    mx = jnp.max(e2, axis=1, keepdims=True)
    p = jnp.exp(e2 - mx)
    att2 = p / jnp.sum(p, axis=1, keepdims=True)
    nd = jnp.dot(att2, edge_ref[:, :], preferred_element_type=jnp.float32)
    out_ref[:, :] = jnp.where(nd > 0, nd, jnp.exp(nd) - 1.0)


def _node_phase(H, t, u_row, edge):
    grid = (N // _BN,)
    return pl.pallas_call(
        _node_kernel,
        grid=grid,
        in_specs=[
            pl.BlockSpec((_BN, M), lambda i: (i, 0)),
            pl.BlockSpec((_BN, 1), lambda i: (i, 0)),
            pl.BlockSpec((1, M), lambda i: (0, 0)),
            pl.BlockSpec((M, D), lambda i: (0, 0)),
        ],
        out_specs=pl.BlockSpec((_BN, D), lambda i: (i, 0)),
        out_shape=jax.ShapeDtypeStruct((N, D), jnp.float32),
    )(H, t, u_row, edge)


def _node_lin_kernel(H_ref, t_ref, u_ref, edge_ref, linW_ref, linb_ref, out_ref):
    z = t_ref[:, :] + u_ref[:, :]
    z = jnp.where(z >= 0, z, 0.2 * z)
    e2 = jnp.where(H_ref[:, :] > 0, z, NEG)
    mx = jnp.max(e2, axis=1, keepdims=True)
    p = jnp.exp(e2 - mx)
    att2 = p / jnp.sum(p, axis=1, keepdims=True)
    nd = jnp.dot(att2, edge_ref[:, :], preferred_element_type=jnp.float32)
    h = jnp.where(nd > 0, nd, jnp.exp(nd) - 1.0)
    out_ref[:, :] = jax.lax.dot_general(
        h, linW_ref[:, :], (((1,), (1,)), ((), ())),
        preferred_element_type=jnp.float32) + linb_ref[:, :]


def _node_phase_lin(H, t, u_row, edge, lin_W, lin_b):
    ncat = lin_W.shape[0]
    grid = (N // _BN,)
    return pl.pallas_call(
        _node_lin_kernel,
        grid=grid,
        in_specs=[
            pl.BlockSpec((_BN, M), lambda i: (i, 0)),
            pl.BlockSpec((_BN, 1), lambda i: (i, 0)),
            pl.BlockSpec((1, M), lambda i: (0, 0)),
            pl.BlockSpec((M, D), lambda i: (0, 0)),
            pl.BlockSpec((ncat, D), lambda i: (0, 0)),
            pl.BlockSpec((1, ncat), lambda i: (0, 0)),
        ],
        out_specs=pl.BlockSpec((_BN, ncat), lambda i: (i, 0)),
        out_shape=jax.ShapeDtypeStruct((N, ncat), jnp.float32),
    )(H, t, u_row, edge, lin_W, lin_b.reshape(1, ncat))


def kernel(items, H, pair_r, pair_c, emb, W2_1, W3_1, a_1, a2_1, wc_1,
           W_2, W2_2, W3_2, a_2, a2_2, wc_2, lin_W, lin_b):
    x0 = _gather_rows_sc(emb, items)

    ls1, t1, v3T1 = _prep(x0, W2_1, W3_1, a_1, a2_1, wc_1)
    edge1, u1 = _edge_phase(H, ls1, x0, v3T1)
    h1 = _node_phase(H, t1, u1.reshape(1, M), edge1)

    ls2, t2, v3T2, xw = _prep2(h1, W2_2, W3_2, a_2, a2_2, wc_2, W_2)
    edge2, u2 = _edge_phase(H, ls2, xw, v3T2)
    out = _node_phase_lin(H, t2, u2.reshape(1, M), edge2, lin_W, lin_b)
    return out


# speedup vs baseline: 17.3315x; 1.1863x over previous
import functools

import jax
import jax.numpy as jnp
from jax.experimental import pallas as pl
from jax.experimental.pallas import tpu as pltpu
from jax.experimental.pallas import tpu_sc as plsc

N = 4096
M = 2048
D = 256
NEG = float(-9e15)
_BM = 256
_BN = 512
_NEB = M // _BM
_NNB = N // _BN


def _gather_rows_sc(emb, items):
    info = plsc.get_sparse_core_info()
    nw = info.num_cores * info.num_subcores
    b = items.shape[0]
    d = emb.shape[1]
    b_per_w = b // nw
    mesh = plsc.VectorSubcoreMesh(core_axis_name="c", subcore_axis_name="s")

    @functools.partial(
        pl.kernel, mesh=mesh,
        out_type=jax.ShapeDtypeStruct((b, d), emb.dtype),
        scratch_types=[
            pltpu.VMEM((b_per_w,), jnp.int32),
            pltpu.VMEM((b_per_w, d), jnp.float32),
            pltpu.SemaphoreType.DMA,
        ],
    )
    def k(emb_hbm, items_hbm, out_hbm, idx_v, rows_v, sem):
        wid = jax.lax.axis_index("s") * info.num_cores + jax.lax.axis_index("c")
        base = wid * b_per_w
        pltpu.sync_copy(items_hbm.at[pl.ds(base, b_per_w)], idx_v)
        pltpu.async_copy(emb_hbm.at[idx_v], rows_v, sem).wait()
        pltpu.sync_copy(rows_v, out_hbm.at[pl.ds(base, b_per_w)])

    return k(emb, items)


def _make_layer_kernel(with_w, with_lin):
    def body(Ha_ref, Hb_ref, x_ref, W2_ref, W3_ref, a_ref, a2_ref, wc_ref,
             *rest):
        idx = 0
        W_ref = None
        linW_ref = linb_ref = None
        if with_w:
            W_ref = rest[idx]; idx += 1
        if with_lin:
            linW_ref = rest[idx]; idx += 1
            linb_ref = rest[idx]; idx += 1
        out_ref = rest[idx]; idx += 1
        ls_sc, t_sc, v3T_sc, edge_sc, u_sc = rest[idx:idx + 5]
        if with_w:
            xa_sc = rest[idx + 5]
        else:
            xa_sc = None

        i = pl.program_id(0)

        @pl.when(i == 0)
        def _prep():
            a_top = a_ref[:D, :]
            a_bot = a_ref[D:, :]
            a2_top = a2_ref[:D, :]
            a2_bot = a2_ref[D:, :]
            c0 = jnp.sum(wc_ref[:, :] * a_top)
            v_s = jnp.dot(W2_ref[:, :], a_bot, preferred_element_type=jnp.float32)
            v_t = jnp.dot(W2_ref[:, :], a2_top, preferred_element_type=jnp.float32)
            s = jnp.dot(x_ref[:, :], v_s, preferred_element_type=jnp.float32) + c0
            ls_sc[:, :] = jnp.where(s >= 0, s, 0.2 * s)
            t_sc[:, :] = jnp.dot(x_ref[:, :], v_t, preferred_element_type=jnp.float32)
            v3T_sc[:, :] = jax.lax.dot_general(
                a2_bot, W3_ref[:, :], (((0,), (1,)), ((), ())),
                preferred_element_type=jnp.float32)
            if with_w:
                xa_sc[:, :] = jnp.dot(x_ref[:, :], W_ref[:, :],
                                      preferred_element_type=jnp.float32)

        @pl.when((i >= 1) & (i <= _NEB))
        def _edge():
            blk = i - 1
            h = Ha_ref[:, :]
            e = jnp.where(h > 0, ls_sc[:, :], NEG)
            mx = jnp.max(e, axis=0, keepdims=True)
            p = jnp.exp(e - mx)
            att = p / jnp.sum(p, axis=0, keepdims=True)
            xa = xa_sc[:, :] if with_w else x_ref[:, :]
            edge_sc[pl.ds(blk * _BM, _BM), :] = jax.lax.dot_general(
                att.astype(jnp.bfloat16), xa.astype(jnp.bfloat16),
                (((0,), (0,)), ((), ())),
                preferred_element_type=jnp.float32)

        @pl.when(i == _NEB + 1)
        def _u():
            u_sc[:, :] = jax.lax.dot_general(
                v3T_sc[:, :], edge_sc[:, :], (((1,), (1,)), ((), ())),
                preferred_element_type=jnp.float32)

        @pl.when(i >= _NEB + 1)
        def _node():
            blk = i - (_NEB + 1)
            tb = t_sc[pl.ds(blk * _BN, _BN), :]
            z = tb + u_sc[:, :]
            z = jnp.where(z >= 0, z, 0.2 * z)
            e2 = jnp.where(Hb_ref[:, :] > 0, z, NEG)
            mx = jnp.max(e2, axis=1, keepdims=True)
            p = jnp.exp(e2 - mx)
            att2 = p / jnp.sum(p, axis=1, keepdims=True)
            nd = jnp.dot(att2.astype(jnp.bfloat16),
                         edge_sc[:, :].astype(jnp.bfloat16),
                         preferred_element_type=jnp.float32)
            hh = jnp.where(nd > 0, nd, jnp.exp(nd) - 1.0)
            if with_lin:
                out_ref[:, :] = jax.lax.dot_general(
                    hh, linW_ref[:, :], (((1,), (1,)), ((), ())),
                    preferred_element_type=jnp.float32) + linb_ref[:, :]
            else:
                out_ref[:, :] = hh

    return body


def _layer(H, x, W2, W3, a, a2, wc, W=None, lin_W=None, lin_b=None):
    with_w = W is not None
    with_lin = lin_W is not None
    ncat = lin_W.shape[0] if with_lin else D
    grid = (1 + _NEB + _NNB,)

    def ha_map(i):
        return (0, jnp.clip(i - 1, 0, _NEB - 1))

    def hb_map(i):
        return (jnp.clip(i - (_NEB + 1), 0, _NNB - 1), 0)

    def out_map(i):
        return (jnp.clip(i - (_NEB + 1), 0, _NNB - 1), 0)

    in_specs = [
        pl.BlockSpec((N, _BM), ha_map),
        pl.BlockSpec((_BN, M), hb_map),
        pl.BlockSpec((N, D), lambda i: (0, 0)),
        pl.BlockSpec((D, D), lambda i: (0, 0)),
        pl.BlockSpec((D, D), lambda i: (0, 0)),
        pl.BlockSpec((2 * D, 1), lambda i: (0, 0)),
        pl.BlockSpec((2 * D, 1), lambda i: (0, 0)),
        pl.BlockSpec((D, 1), lambda i: (0, 0)),
    ]
    args = [H, H, x, W2, W3, a, a2, wc.reshape(D, 1)]
    if with_w:
        in_specs.append(pl.BlockSpec((D, D), lambda i: (0, 0)))
        args.append(W)
    if with_lin:
        in_specs.append(pl.BlockSpec((ncat, D), lambda i: (0, 0)))
        in_specs.append(pl.BlockSpec((1, ncat), lambda i: (0, 0)))
        args.append(lin_W)
        args.append(lin_b.reshape(1, ncat))

    scratch_shapes = [
        pltpu.VMEM((N, 1), jnp.float32),
        pltpu.VMEM((N, 1), jnp.float32),
        pltpu.VMEM((1, D), jnp.float32),
        pltpu.VMEM((M, D), jnp.float32),
        pltpu.VMEM((1, M), jnp.float32),
    ]
    if with_w:
        scratch_shapes.append(pltpu.VMEM((N, D), jnp.float32))

    return pl.pallas_call(
        _make_layer_kernel(with_w, with_lin),
        grid=grid,
        in_specs=in_specs,
        out_specs=pl.BlockSpec((_BN, ncat), out_map),
        out_shape=jax.ShapeDtypeStruct((N, ncat), jnp.float32),
        scratch_shapes=scratch_shapes,
    )(*args)


def kernel(items, H, pair_r, pair_c, emb, W2_1, W3_1, a_1, a2_1, wc_1,
           W_2, W2_2, W3_2, a_2, a2_2, wc_2, lin_W, lin_b):
    x0 = _gather_rows_sc(emb, items)
    h1 = _layer(H, x0, W2_1, W3_1, a_1, a2_1, wc_1)
    out = _layer(H, h1, W2_2, W3_2, a_2, a2_2, wc_2,
                 W=W_2, lin_W=lin_W, lin_b=lin_b)
    return out


# speedup vs baseline: 19.5782x; 1.1296x over previous
import functools

import jax
import jax.numpy as jnp
from jax.experimental import pallas as pl
from jax.experimental.pallas import tpu as pltpu
from jax.experimental.pallas import tpu_sc as plsc

N = 4096
M = 2048
D = 256
NEG = float(-9e15)
_BM = 256
_BN = 512
_NEB = M // _BM
_NNB = N // _BN


def _gather_rows_sc(emb, items):
    info = plsc.get_sparse_core_info()
    nw = info.num_cores * info.num_subcores
    b = items.shape[0]
    d = emb.shape[1]
    b_per_w = b // nw
    mesh = plsc.VectorSubcoreMesh(core_axis_name="c", subcore_axis_name="s")

    @functools.partial(
        pl.kernel, mesh=mesh,
        out_type=jax.ShapeDtypeStruct((b, d), emb.dtype),
        scratch_types=[
            pltpu.VMEM((b_per_w,), jnp.int32),
            pltpu.VMEM((b_per_w, d), jnp.float32),
            pltpu.SemaphoreType.DMA,
        ],
    )
    def k(emb_hbm, items_hbm, out_hbm, idx_v, rows_v, sem):
        wid = jax.lax.axis_index("s") * info.num_cores + jax.lax.axis_index("c")
        base = wid * b_per_w
        pltpu.sync_copy(items_hbm.at[pl.ds(base, b_per_w)], idx_v)
        pltpu.async_copy(emb_hbm.at[idx_v], rows_v, sem).wait()
        pltpu.sync_copy(rows_v, out_hbm.at[pl.ds(base, b_per_w)])

    return k(emb, items)


def _make_layer_kernel(with_w, with_lin):
    def body(Ha_ref, Hb_ref, x_ref, W2_ref, W3_ref, a_ref, a2_ref, wc_ref,
             *rest):
        idx = 0
        W_ref = None
        linW_ref = linb_ref = None
        if with_w:
            W_ref = rest[idx]; idx += 1
        if with_lin:
            linW_ref = rest[idx]; idx += 1
            linb_ref = rest[idx]; idx += 1
        out_ref = rest[idx]; idx += 1
        ls_sc, t_sc, v3T_sc, edge_sc, u_sc, gz_sc, xa_sc, me_sc = rest[idx:idx + 8]

        i = pl.program_id(0)

        @pl.when(i == 0)
        def _prep():
            a_top = a_ref[:D, :]
            a_bot = a_ref[D:, :]
            a2_top = a2_ref[:D, :]
            a2_bot = a2_ref[D:, :]
            c0 = jnp.sum(wc_ref[:, :] * a_top)
            v_st = jnp.dot(W2_ref[:, :], jnp.concatenate([a_bot, a2_top], axis=1),
                           preferred_element_type=jnp.float32)
            st = jnp.dot(x_ref[:, :], v_st, preferred_element_type=jnp.float32)
            s = st[:, 0:1] + c0
            ls = jnp.where(s >= 0, s, 0.2 * s)
            gmax = jnp.max(ls)
            ls_sc[:, :] = jnp.exp(jnp.maximum(ls - gmax, -80.0))
            t_sc[:, :] = st[:, 1:2]
            v3T_sc[:, :] = jax.lax.dot_general(
                a2_bot, W3_ref[:, :], (((0,), (1,)), ((), ())),
                preferred_element_type=jnp.float32)
            if with_w:
                xa_sc[:, :D] = jnp.dot(
                    x_ref[:, :], W_ref[:, :],
                    preferred_element_type=jnp.float32).astype(jnp.bfloat16)
            else:
                xa_sc[:, :D] = x_ref[:, :].astype(jnp.bfloat16)
            xa_sc[:, D:] = jnp.ones((N, 128), jnp.bfloat16)

        @pl.when((i >= 1) & (i <= _NEB))
        def _edge():
            blk = i - 1
            p = jnp.where(Ha_ref[:, :] > 0, ls_sc[:, :], 0.0).astype(jnp.bfloat16)
            er = jax.lax.dot_general(
                p, xa_sc[:, :], (((0,), (0,)), ((), ())),
                preferred_element_type=jnp.float32)
            edge_sc[pl.ds(blk * _BM, _BM), :D] = (
                er[:, :D] / er[:, D:D + 1]).astype(jnp.bfloat16)

        @pl.when(i == _NEB + 1)
        def _u():
            edge_sc[:, D:] = jnp.ones((M, 128), jnp.bfloat16)
            onesM = jnp.ones((1, M), jnp.bfloat16)
            me_sc[:, :] = jnp.dot(onesM, edge_sc[:, :D],
                                  preferred_element_type=jnp.float32) / M
            u = jax.lax.dot_general(
                v3T_sc[:, :].astype(jnp.bfloat16), edge_sc[:, :D],
                (((1,), (1,)), ((), ())),
                preferred_element_type=jnp.float32)
            u_sc[:, :] = u
            tu = jnp.max(t_sc[:, :]) + jnp.max(u)
            gz_sc[0, 0] = jnp.where(tu >= 0, tu, 0.2 * tu)

        @pl.when(i >= _NEB + 1)
        def _node():
            blk = i - (_NEB + 1)
            tb = t_sc[pl.ds(blk * _BN, _BN), :]
            z = tb + u_sc[:, :]
            z = jnp.where(z >= 0, z, 0.2 * z)
            pz = jnp.exp(jnp.maximum(z - gz_sc[0, 0], -80.0))
            p = jnp.where(Hb_ref[:, :] > 0, pz, 0.0).astype(jnp.bfloat16)
            ndr = jnp.dot(p, edge_sc[:, :],
                          preferred_element_type=jnp.float32)
            sm = ndr[:, D:D + 1]
            nd = jnp.where(sm > 0, ndr[:, :D] / jnp.where(sm > 0, sm, 1.0),
                           me_sc[:, :])
            hh = jnp.where(nd > 0, nd, jnp.exp(nd) - 1.0)
            if with_lin:
                out_ref[:, :] = jax.lax.dot_general(
                    hh, linW_ref[:, :], (((1,), (1,)), ((), ())),
                    preferred_element_type=jnp.float32) + linb_ref[:, :]
            else:
                out_ref[:, :] = hh

    return body


def _layer(H, x, W2, W3, a, a2, wc, W=None, lin_W=None, lin_b=None):
    with_w = W is not None
    with_lin = lin_W is not None
    ncat = lin_W.shape[0] if with_lin else D
    grid = (1 + _NEB + _NNB,)

    def ha_map(i):
        return (0, jnp.clip(i - 1, 0, _NEB - 1))

    def hb_map(i):
        return (jnp.clip(i - (_NEB + 1), 0, _NNB - 1), 0)

    def out_map(i):
        return (jnp.clip(i - (_NEB + 1), 0, _NNB - 1), 0)

    in_specs = [
        pl.BlockSpec((N, _BM), ha_map),
        pl.BlockSpec((_BN, M), hb_map),
        pl.BlockSpec((N, D), lambda i: (0, 0)),
        pl.BlockSpec((D, D), lambda i: (0, 0)),
        pl.BlockSpec((D, D), lambda i: (0, 0)),
        pl.BlockSpec((2 * D, 1), lambda i: (0, 0)),
        pl.BlockSpec((2 * D, 1), lambda i: (0, 0)),
        pl.BlockSpec((D, 1), lambda i: (0, 0)),
    ]
    args = [H, H, x, W2, W3, a, a2, wc.reshape(D, 1)]
    if with_w:
        in_specs.append(pl.BlockSpec((D, D), lambda i: (0, 0)))
        args.append(W)
    if with_lin:
        in_specs.append(pl.BlockSpec((ncat, D), lambda i: (0, 0)))
        in_specs.append(pl.BlockSpec((1, ncat), lambda i: (0, 0)))
        args.append(lin_W)
        args.append(lin_b.reshape(1, ncat))

    scratch_shapes = [
        pltpu.VMEM((N, 1), jnp.float32),
        pltpu.VMEM((N, 1), jnp.float32),
        pltpu.VMEM((1, D), jnp.float32),
        pltpu.VMEM((M, D + 128), jnp.bfloat16),
        pltpu.VMEM((1, M), jnp.float32),
        pltpu.SMEM((1, 1), jnp.float32),
        pltpu.VMEM((N, D + 128), jnp.bfloat16),
        pltpu.VMEM((1, D), jnp.float32),
    ]

    return pl.pallas_call(
        _make_layer_kernel(with_w, with_lin),
        grid=grid,
        in_specs=in_specs,
        out_specs=pl.BlockSpec((_BN, ncat), out_map),
        out_shape=jax.ShapeDtypeStruct((N, ncat), jnp.float32),
        scratch_shapes=scratch_shapes,
    )(*args)


def kernel(items, H, pair_r, pair_c, emb, W2_1, W3_1, a_1, a2_1, wc_1,
           W_2, W2_2, W3_2, a_2, a2_2, wc_2, lin_W, lin_b):
    x0 = _gather_rows_sc(emb, items)
    h1 = _layer(H, x0, W2_1, W3_1, a_1, a2_1, wc_1)
    out = _layer(H, h1, W2_2, W3_2, a_2, a2_2, wc_2,
                 W=W_2, lin_W=lin_W, lin_b=lin_b)
    return out


# speedup vs baseline: 19.9451x; 1.0187x over previous
import functools

import jax
import jax.numpy as jnp
from jax.experimental import pallas as pl
from jax.experimental.pallas import tpu as pltpu
from jax.experimental.pallas import tpu_sc as plsc

N = 4096
M = 2048
D = 256
DE = D + 128
_BM = 256
_BN = 512
_NEB = M // _BM
_NNB = N // _BN
_N1 = 1 + _NEB
_P2 = _N1 + _NNB
_E2 = _P2 + 1
_N2 = _E2 + _NEB


def _gather_rows_sc(emb, items):
    info = plsc.get_sparse_core_info()
    nw = info.num_cores * info.num_subcores
    b = items.shape[0]
    d = emb.shape[1]
    b_per_w = b // nw
    mesh = plsc.VectorSubcoreMesh(core_axis_name="c", subcore_axis_name="s")

    @functools.partial(
        pl.kernel, mesh=mesh,
        out_type=jax.ShapeDtypeStruct((b, d), emb.dtype),
        scratch_types=[
            pltpu.VMEM((b_per_w,), jnp.int32),
            pltpu.VMEM((b_per_w, d), jnp.float32),
            pltpu.SemaphoreType.DMA,
        ],
    )
    def k(emb_hbm, items_hbm, out_hbm, idx_v, rows_v, sem):
        wid = jax.lax.axis_index("s") * info.num_cores + jax.lax.axis_index("c")
        base = wid * b_per_w
        pltpu.sync_copy(items_hbm.at[pl.ds(base, b_per_w)], idx_v)
        pltpu.async_copy(emb_hbm.at[idx_v], rows_v, sem).wait()
        pltpu.sync_copy(rows_v, out_hbm.at[pl.ds(base, b_per_w)])

    return k(emb, items)


def _fwd_kernel(Ha_ref, Hb_ref, x_ref,
                W2a_ref, W3a_ref, aa_ref, a2a_ref, wca_ref,
                W2b_ref, W3b_ref, ab_ref, a2b_ref, wcb_ref, W_ref,
                linW_ref, linb_ref,
                out_ref,
                ls_sc, t_sc, v3T_sc, edge_sc, u_sc, gz_sc, xa_sc, me_sc,
                h1_sc):
    i = pl.program_id(0)

    def prep(x, W2_ref, W3_ref, a_ref, a2_ref, wc_ref, Wt_ref):
        a_top = a_ref[:D, :]
        a_bot = a_ref[D:, :]
        a2_top = a2_ref[:D, :]
        a2_bot = a2_ref[D:, :]
        c0 = jnp.sum(wc_ref[:, :] * a_top)
        v_st = jnp.dot(W2_ref[:, :],
                       jnp.concatenate([a_bot, a2_top], axis=1),
                       preferred_element_type=jnp.float32)
        st = jnp.dot(x, v_st, preferred_element_type=jnp.float32)
        s = st[:, 0:1] + c0
        ls = jnp.where(s >= 0, s, 0.2 * s)
        gmax = jnp.max(ls)
        ls_sc[:, :] = jnp.exp(jnp.maximum(ls - gmax, -80.0))
        t_sc[:, :] = st[:, 1:2]
        v3T_sc[:, :] = jax.lax.dot_general(
            a2_bot, W3_ref[:, :], (((0,), (1,)), ((), ())),
            preferred_element_type=jnp.float32)
        if Wt_ref is not None:
            xa_sc[:, :D] = jnp.dot(
                x, Wt_ref[:, :],
                preferred_element_type=jnp.float32).astype(jnp.bfloat16)
        else:
            xa_sc[:, :D] = x.astype(jnp.bfloat16)
        xa_sc[:, D:] = jnp.ones((N, 128), jnp.bfloat16)

    def edge_step(blk):
        p = (Ha_ref[:, :] * ls_sc[:, :]).astype(jnp.bfloat16)
        er = jax.lax.dot_general(
            p, xa_sc[:, :], (((0,), (0,)), ((), ())),
            preferred_element_type=jnp.float32)
        edge_sc[pl.ds(blk * _BM, _BM), :D] = (
            er[:, :D] / er[:, D:D + 1]).astype(jnp.bfloat16)

    def u_step():
        edge_sc[:, D:] = jnp.ones((M, 128), jnp.bfloat16)
        onesM = jnp.ones((1, M), jnp.bfloat16)
        me_sc[:, :] = jnp.dot(onesM, edge_sc[:, :D],
                              preferred_element_type=jnp.float32) / M
        u = jax.lax.dot_general(
            v3T_sc[:, :].astype(jnp.bfloat16), edge_sc[:, :D],
            (((1,), (1,)), ((), ())),
            preferred_element_type=jnp.float32)
        u_sc[:, :] = u
        tu = jnp.max(t_sc[:, :]) + jnp.max(u)
        gz_sc[0, 0] = jnp.where(tu >= 0, tu, 0.2 * tu)

    def node_step(blk):
        tb = t_sc[pl.ds(blk * _BN, _BN), :]
        z = tb + u_sc[:, :]
        z = jnp.where(z >= 0, z, 0.2 * z)
        pz = jnp.exp(jnp.maximum(z - gz_sc[0, 0], -80.0))
        p = (Hb_ref[:, :] * pz).astype(jnp.bfloat16)
        ndr = jnp.dot(p, edge_sc[:, :],
                      preferred_element_type=jnp.float32)
        sm = ndr[:, D:D + 1]
        nd = jnp.where(sm > 0, ndr[:, :D] / jnp.where(sm > 0, sm, 1.0),
                       me_sc[:, :])
        return jnp.where(nd > 0, nd, jnp.exp(nd) - 1.0)

    @pl.when(i == 0)
    def _prep1():
        prep(x_ref[:, :], W2a_ref, W3a_ref, aa_ref, a2a_ref, wca_ref, None)

    @pl.when((i >= 1) & (i < _N1))
    def _edge1():
        edge_step(i - 1)

    @pl.when(i == _N1)
    def _u1():
        u_step()

    @pl.when((i >= _N1) & (i < _P2))
    def _node1():
        blk = i - _N1
        h1_sc[pl.ds(blk * _BN, _BN), :] = node_step(blk)

    @pl.when(i == _P2)
    def _prep2():
        prep(h1_sc[:, :], W2b_ref, W3b_ref, ab_ref, a2b_ref, wcb_ref, W_ref)

    @pl.when((i >= _E2) & (i < _N2))
    def _edge2():
        edge_step(i - _E2)

    @pl.when(i == _N2)
    def _u2():
        u_step()

    @pl.when(i >= _N2)
    def _node2():
        blk = i - _N2
        hh = node_step(blk)
        out_ref[:, :] = jax.lax.dot_general(
            hh, linW_ref[:, :], (((1,), (1,)), ((), ())),
            preferred_element_type=jnp.float32) + linb_ref[:, :]


def kernel(items, H, pair_r, pair_c, emb, W2_1, W3_1, a_1, a2_1, wc_1,
           W_2, W2_2, W3_2, a_2, a2_2, wc_2, lin_W, lin_b):
    x0 = _gather_rows_sc(emb, items)
    ncat = lin_W.shape[0]
    grid = (_N2 + _NNB,)

    def ha_map(i):
        b = jnp.where(i >= _E2, i - _E2, i - 1)
        return (0, jnp.clip(b, 0, _NEB - 1))

    def hb_map(i):
        b = jnp.where(i >= _N2, i - _N2, i - _N1)
        return (jnp.clip(b, 0, _NNB - 1), 0)

    def out_map(i):
        return (jnp.clip(i - _N2, 0, _NNB - 1), 0)

    def full(i):
        return (0, 0)

    in_specs = [
        pl.BlockSpec((N, _BM), ha_map),
        pl.BlockSpec((_BN, M), hb_map),
        pl.BlockSpec((N, D), full),
        pl.BlockSpec((D, D), full),
        pl.BlockSpec((D, D), full),
        pl.BlockSpec((2 * D, 1), full),
        pl.BlockSpec((2 * D, 1), full),
        pl.BlockSpec((D, 1), full),
        pl.BlockSpec((D, D), full),
        pl.BlockSpec((D, D), full),
        pl.BlockSpec((2 * D, 1), full),
        pl.BlockSpec((2 * D, 1), full),
        pl.BlockSpec((D, 1), full),
        pl.BlockSpec((D, D), full),
        pl.BlockSpec((ncat, D), full),
        pl.BlockSpec((1, ncat), full),
    ]
    scratch_shapes = [
        pltpu.VMEM((N, 1), jnp.float32),
        pltpu.VMEM((N, 1), jnp.float32),
        pltpu.VMEM((1, D), jnp.float32),
        pltpu.VMEM((M, DE), jnp.bfloat16),
        pltpu.VMEM((1, M), jnp.float32),
        pltpu.SMEM((1, 1), jnp.float32),
        pltpu.VMEM((N, DE), jnp.bfloat16),
        pltpu.VMEM((1, D), jnp.float32),
        pltpu.VMEM((N, D), jnp.float32),
    ]
    return pl.pallas_call(
        _fwd_kernel,
        grid=grid,
        in_specs=in_specs,
        out_specs=pl.BlockSpec((_BN, ncat), out_map),
        out_shape=jax.ShapeDtypeStruct((N, ncat), jnp.float32),
        scratch_shapes=scratch_shapes,
    )(H, H, x0, W2_1, W3_1, a_1, a2_1, wc_1.reshape(D, 1),
      W2_2, W3_2, a_2, a2_2, wc_2.reshape(D, 1), W_2,
      lin_W, lin_b.reshape(1, ncat))


# speedup vs baseline: 20.6593x; 1.0358x over previous
import functools

import jax
import jax.numpy as jnp
from jax.experimental import pallas as pl
from jax.experimental.pallas import tpu as pltpu
from jax.experimental.pallas import tpu_sc as plsc

N = 4096
M = 2048
D = 256
DE = D + 128
_BM = 256
_BN = 512
_NEB = M // _BM
_NNB = N // _BN
_N1 = 1 + _NEB
_P2 = _N1 + _NNB
_E2 = _P2 + 1
_N2 = _E2 + _NEB


def _gather_rows_sc(emb, items):
    info = plsc.get_sparse_core_info()
    nw = info.num_cores * info.num_subcores
    b = items.shape[0]
    d = emb.shape[1]
    b_per_w = b // nw
    mesh = plsc.VectorSubcoreMesh(core_axis_name="c", subcore_axis_name="s")

    @functools.partial(
        pl.kernel, mesh=mesh,
        out_type=jax.ShapeDtypeStruct((b, d), emb.dtype),
        scratch_types=[
            pltpu.VMEM((b_per_w,), jnp.int32),
            pltpu.VMEM((b_per_w, d), jnp.float32),
            pltpu.SemaphoreType.DMA,
        ],
    )
    def k(emb_hbm, items_hbm, out_hbm, idx_v, rows_v, sem):
        wid = jax.lax.axis_index("s") * info.num_cores + jax.lax.axis_index("c")
        base = wid * b_per_w
        pltpu.sync_copy(items_hbm.at[pl.ds(base, b_per_w)], idx_v)
        pltpu.async_copy(emb_hbm.at[idx_v], rows_v, sem).wait()
        pltpu.sync_copy(rows_v, out_hbm.at[pl.ds(base, b_per_w)])

    return k(emb, items)


def _fwd_kernel(Ha_ref, Hb_ref, x_ref,
                W2a_ref, W3a_ref, aa_ref, a2a_ref, wca_ref,
                W2b_ref, W3b_ref, ab_ref, a2b_ref, wcb_ref, W_ref,
                linW_ref, linb_ref,
                out_ref,
                ls_sc, t_sc, v3T_sc, edge_sc, u_sc, gz_sc, xa_sc, me_sc,
                h1_sc):
    i = pl.program_id(0)

    def prep(x, W2_ref, W3_ref, a_ref, a2_ref, wc_ref, Wt_ref):
        a_top = a_ref[:D, :]
        a_bot = a_ref[D:, :]
        a2_top = a2_ref[:D, :]
        a2_bot = a2_ref[D:, :]
        c0 = jnp.sum(wc_ref[:, :] * a_top)
        v_st = jnp.dot(W2_ref[:, :],
                       jnp.concatenate([a_bot, a2_top], axis=1),
                       preferred_element_type=jnp.float32)
        st = jnp.dot(x, v_st, preferred_element_type=jnp.float32)
        s = st[:, 0:1] + c0
        ls = jnp.where(s >= 0, s, 0.2 * s)
        gmax = jnp.max(ls)
        ls_sc[:, :] = jnp.exp(jnp.maximum(ls - gmax, -80.0))
        t_sc[:, :] = st[:, 1:2]
        v3T_sc[:, :] = jax.lax.dot_general(
            a2_bot, W3_ref[:, :], (((0,), (1,)), ((), ())),
            preferred_element_type=jnp.float32)
        if Wt_ref is not None:
            xa_sc[:, :D] = jnp.dot(
                x, Wt_ref[:, :],
                preferred_element_type=jnp.float32).astype(jnp.bfloat16)
        else:
            xa_sc[:, :D] = x.astype(jnp.bfloat16)
        xa_sc[:, D:] = jnp.ones((N, 128), jnp.bfloat16)

    def edge_step(blk):
        p = jnp.where(Ha_ref[:, :] > 0, ls_sc[:, :], 0.0).astype(jnp.bfloat16)
        er = jax.lax.dot_general(
            p, xa_sc[:, :], (((0,), (0,)), ((), ())),
            preferred_element_type=jnp.float32)
        edge_sc[pl.ds(blk * _BM, _BM), :D] = (
            er[:, :D] / er[:, D:D + 1]).astype(jnp.bfloat16)

    def u_step():
        edge_sc[:, D:] = jnp.ones((M, 128), jnp.bfloat16)
        onesM = jnp.ones((1, M), jnp.bfloat16)
        me_sc[:, :] = jnp.dot(onesM, edge_sc[:, :D],
                              preferred_element_type=jnp.float32) / M
        u = jax.lax.dot_general(
            v3T_sc[:, :].astype(jnp.bfloat16), edge_sc[:, :D],
            (((1,), (1,)), ((), ())),
            preferred_element_type=jnp.float32)
        u_sc[:, :] = u
        tu = jnp.max(t_sc[:, :]) + jnp.max(u)
        gz_sc[0, 0] = jnp.where(tu >= 0, tu, 0.2 * tu)

    def node_step(blk):
        tb = t_sc[pl.ds(blk * _BN, _BN), :]
        z = tb + u_sc[:, :]
        z = jnp.where(z >= 0, z, 0.2 * z)
        pz = jnp.exp(jnp.maximum(z - gz_sc[0, 0], -80.0))
        p = jnp.where(Hb_ref[:, :] > 0, pz, 0.0).astype(jnp.bfloat16)
        ndr = jnp.dot(p, edge_sc[:, :],
                      preferred_element_type=jnp.float32)
        sm = ndr[:, D:D + 1]
        nd = jnp.where(sm > 0, ndr[:, :D] / jnp.where(sm > 0, sm, 1.0),
                       me_sc[:, :])
        return jnp.where(nd > 0, nd, jnp.exp(nd) - 1.0)

    @pl.when(i == 0)
    def _prep1():
        prep(x_ref[:, :], W2a_ref, W3a_ref, aa_ref, a2a_ref, wca_ref, None)

    @pl.when((i >= 1) & (i < _N1))
    def _edge1():
        edge_step(i - 1)

    @pl.when(i == _N1)
    def _u1():
        u_step()

    @pl.when((i >= _N1) & (i < _P2))
    def _node1():
        blk = i - _N1
        h1_sc[pl.ds(blk * _BN, _BN), :] = node_step(blk)

    @pl.when(i == _P2)
    def _prep2():
        prep(h1_sc[:, :], W2b_ref, W3b_ref, ab_ref, a2b_ref, wcb_ref, W_ref)

    @pl.when((i >= _E2) & (i < _N2))
    def _edge2():
        edge_step(i - _E2)

    @pl.when(i == _N2)
    def _u2():
        u_step()

    @pl.when(i >= _N2)
    def _node2():
        blk = i - _N2
        hh = node_step(blk)
        out_ref[:, :] = jax.lax.dot_general(
            hh, linW_ref[:, :], (((1,), (1,)), ((), ())),
            preferred_element_type=jnp.float32) + linb_ref[:, :]


def kernel(items, H, pair_r, pair_c, emb, W2_1, W3_1, a_1, a2_1, wc_1,
           W_2, W2_2, W3_2, a_2, a2_2, wc_2, lin_W, lin_b):
    x0 = _gather_rows_sc(emb, items)
    ncat = lin_W.shape[0]
    grid = (_N2 + _NNB,)

    def ha_map(i):
        b = jnp.where(i >= _E2, i - _E2, i - 1)
        return (0, jnp.clip(b, 0, _NEB - 1))

    def hb_map(i):
        b = jnp.where(i >= _N2, i - _N2, i - _N1)
        return (jnp.clip(b, 0, _NNB - 1), 0)

    def out_map(i):
        return (jnp.clip(i - _N2, 0, _NNB - 1), 0)

    def full(i):
        return (0, 0)

    in_specs = [
        pl.BlockSpec((N, _BM), ha_map),
        pl.BlockSpec((_BN, M), hb_map),
        pl.BlockSpec((N, D), full),
        pl.BlockSpec((D, D), full),
        pl.BlockSpec((D, D), full),
        pl.BlockSpec((2 * D, 1), full),
        pl.BlockSpec((2 * D, 1), full),
        pl.BlockSpec((D, 1), full),
        pl.BlockSpec((D, D), full),
        pl.BlockSpec((D, D), full),
        pl.BlockSpec((2 * D, 1), full),
        pl.BlockSpec((2 * D, 1), full),
        pl.BlockSpec((D, 1), full),
        pl.BlockSpec((D, D), full),
        pl.BlockSpec((ncat, D), full),
        pl.BlockSpec((1, ncat), full),
    ]
    scratch_shapes = [
        pltpu.VMEM((N, 1), jnp.float32),
        pltpu.VMEM((N, 1), jnp.float32),
        pltpu.VMEM((1, D), jnp.float32),
        pltpu.VMEM((M, DE), jnp.bfloat16),
        pltpu.VMEM((1, M), jnp.float32),
        pltpu.SMEM((1, 1), jnp.float32),
        pltpu.VMEM((N, DE), jnp.bfloat16),
        pltpu.VMEM((1, D), jnp.float32),
        pltpu.VMEM((N, D), jnp.float32),
    ]
    return pl.pallas_call(
        _fwd_kernel,
        grid=grid,
        in_specs=in_specs,
        out_specs=pl.BlockSpec((_BN, ncat), out_map),
        out_shape=jax.ShapeDtypeStruct((N, ncat), jnp.float32),
        scratch_shapes=scratch_shapes,
    )(H, H, x0, W2_1, W3_1, a_1, a2_1, wc_1.reshape(D, 1),
      W2_2, W3_2, a_2, a2_2, wc_2.reshape(D, 1), W_2,
      lin_W, lin_b.reshape(1, ncat))


# speedup vs baseline: 22.2074x; 1.0749x over previous
import functools

import jax
import jax.numpy as jnp
from jax.experimental import pallas as pl
from jax.experimental.pallas import tpu as pltpu
from jax.experimental.pallas import tpu_sc as plsc

N = 4096
M = 2048
D = 256
DE = D + 128
_BM = 512
_BN = 1024
_NEB = M // _BM
_NNB = N // _BN
_N1 = 1 + _NEB
_P2 = _N1 + _NNB
_E2 = _P2 + 1
_N2 = _E2 + _NEB


def _gather_rows_sc(emb, items):
    info = plsc.get_sparse_core_info()
    nw = info.num_cores * info.num_subcores
    b = items.shape[0]
    d = emb.shape[1]
    b_per_w = b // nw
    mesh = plsc.VectorSubcoreMesh(core_axis_name="c", subcore_axis_name="s")

    @functools.partial(
        pl.kernel, mesh=mesh,
        out_type=jax.ShapeDtypeStruct((b, d), emb.dtype),
        scratch_types=[
            pltpu.VMEM((b_per_w,), jnp.int32),
            pltpu.VMEM((b_per_w, d), jnp.float32),
            pltpu.SemaphoreType.DMA,
        ],
    )
    def k(emb_hbm, items_hbm, out_hbm, idx_v, rows_v, sem):
        wid = jax.lax.axis_index("s") * info.num_cores + jax.lax.axis_index("c")
        base = wid * b_per_w
        pltpu.sync_copy(items_hbm.at[pl.ds(base, b_per_w)], idx_v)
        pltpu.async_copy(emb_hbm.at[idx_v], rows_v, sem).wait()
        pltpu.sync_copy(rows_v, out_hbm.at[pl.ds(base, b_per_w)])

    return k(emb, items)


def _fwd_kernel(Ha_ref, Hb_ref, x_ref,
                W2a_ref, W3a_ref, aa_ref, a2a_ref, wca_ref,
                W2b_ref, W3b_ref, ab_ref, a2b_ref, wcb_ref, W_ref,
                linW_ref, linb_ref,
                out_ref,
                ls_sc, t_sc, v3T_sc, edge_sc, u_sc, gz_sc, xa_sc, me_sc,
                h1_sc):
    i = pl.program_id(0)

    def prep(x, W2_ref, W3_ref, a_ref, a2_ref, wc_ref, Wt_ref):
        a_top = a_ref[:D, :]
        a_bot = a_ref[D:, :]
        a2_top = a2_ref[:D, :]
        a2_bot = a2_ref[D:, :]
        c0 = jnp.sum(wc_ref[:, :] * a_top)
        v_st = jnp.dot(W2_ref[:, :],
                       jnp.concatenate([a_bot, a2_top], axis=1),
                       preferred_element_type=jnp.float32)
        st = jnp.dot(x, v_st, preferred_element_type=jnp.float32)
        s = st[:, 0:1] + c0
        ls = jnp.where(s >= 0, s, 0.2 * s)
        gmax = jnp.max(ls)
        ls_sc[:, :] = jnp.exp(jnp.maximum(ls - gmax, -80.0))
        t_sc[:, :] = st[:, 1:2]
        v3T_sc[:, :] = jax.lax.dot_general(
            a2_bot, W3_ref[:, :], (((0,), (1,)), ((), ())),
            preferred_element_type=jnp.float32)
        if Wt_ref is not None:
            xa_sc[:, :D] = jnp.dot(
                x, Wt_ref[:, :],
                preferred_element_type=jnp.float32).astype(jnp.bfloat16)
        else:
            xa_sc[:, :D] = x.astype(jnp.bfloat16)
        xa_sc[:, D:] = jnp.ones((N, 128), jnp.bfloat16)

    def edge_step(blk):
        p = jnp.where(Ha_ref[:, :] > 0, ls_sc[:, :], 0.0).astype(jnp.bfloat16)
        er = jax.lax.dot_general(
            p, xa_sc[:, :], (((0,), (0,)), ((), ())),
            preferred_element_type=jnp.float32)
        edge_sc[pl.ds(blk * _BM, _BM), :D] = (
            er[:, :D] / er[:, D:D + 1]).astype(jnp.bfloat16)

    def u_step():
        edge_sc[:, D:] = jnp.ones((M, 128), jnp.bfloat16)
        onesM = jnp.ones((1, M), jnp.bfloat16)
        me_sc[:, :] = jnp.dot(onesM, edge_sc[:, :D],
                              preferred_element_type=jnp.float32) / M
        u = jax.lax.dot_general(
            v3T_sc[:, :].astype(jnp.bfloat16), edge_sc[:, :D],
            (((1,), (1,)), ((), ())),
            preferred_element_type=jnp.float32)
        u_sc[:, :] = u
        tu = jnp.max(t_sc[:, :]) + jnp.max(u)
        gz_sc[0, 0] = jnp.where(tu >= 0, tu, 0.2 * tu)

    def node_step(blk):
        tb = t_sc[pl.ds(blk * _BN, _BN), :]
        z = tb + u_sc[:, :]
        z = jnp.where(z >= 0, z, 0.2 * z)
        pz = jnp.exp(jnp.maximum(z - gz_sc[0, 0], -80.0))
        p = jnp.where(Hb_ref[:, :] > 0, pz, 0.0).astype(jnp.bfloat16)
        ndr = jnp.dot(p, edge_sc[:, :],
                      preferred_element_type=jnp.float32)
        sm = ndr[:, D:D + 1]
        nd = jnp.where(sm > 0, ndr[:, :D] / jnp.where(sm > 0, sm, 1.0),
                       me_sc[:, :])
        return jnp.where(nd > 0, nd, jnp.exp(nd) - 1.0)

    @pl.when(i == 0)
    def _prep1():
        prep(x_ref[:, :], W2a_ref, W3a_ref, aa_ref, a2a_ref, wca_ref, None)

    @pl.when((i >= 1) & (i < _N1))
    def _edge1():
        edge_step(i - 1)

    @pl.when(i == _N1)
    def _u1():
        u_step()

    @pl.when((i >= _N1) & (i < _P2))
    def _node1():
        blk = i - _N1
        h1_sc[pl.ds(blk * _BN, _BN), :] = node_step(blk)

    @pl.when(i == _P2)
    def _prep2():
        prep(h1_sc[:, :], W2b_ref, W3b_ref, ab_ref, a2b_ref, wcb_ref, W_ref)

    @pl.when((i >= _E2) & (i < _N2))
    def _edge2():
        edge_step(i - _E2)

    @pl.when(i == _N2)
    def _u2():
        u_step()

    @pl.when(i >= _N2)
    def _node2():
        blk = i - _N2
        hh = node_step(blk)
        out_ref[:, :] = jax.lax.dot_general(
            hh, linW_ref[:, :], (((1,), (1,)), ((), ())),
            preferred_element_type=jnp.float32) + linb_ref[:, :]


def kernel(items, H, pair_r, pair_c, emb, W2_1, W3_1, a_1, a2_1, wc_1,
           W_2, W2_2, W3_2, a_2, a2_2, wc_2, lin_W, lin_b):
    x0 = _gather_rows_sc(emb, items)
    ncat = lin_W.shape[0]
    grid = (_N2 + _NNB,)

    def ha_map(i):
        b = jnp.where(i >= _E2, i - _E2, i - 1)
        return (0, jnp.clip(b, 0, _NEB - 1))

    def hb_map(i):
        b = jnp.where(i >= _N2, i - _N2, i - _N1)
        return (jnp.clip(b, 0, _NNB - 1), 0)

    def out_map(i):
        return (jnp.clip(i - _N2, 0, _NNB - 1), 0)

    def full(i):
        return (0, 0)

    in_specs = [
        pl.BlockSpec((N, _BM), ha_map),
        pl.BlockSpec((_BN, M), hb_map),
        pl.BlockSpec((N, D), full),
        pl.BlockSpec((D, D), full),
        pl.BlockSpec((D, D), full),
        pl.BlockSpec((2 * D, 1), full),
        pl.BlockSpec((2 * D, 1), full),
        pl.BlockSpec((D, 1), full),
        pl.BlockSpec((D, D), full),
        pl.BlockSpec((D, D), full),
        pl.BlockSpec((2 * D, 1), full),
        pl.BlockSpec((2 * D, 1), full),
        pl.BlockSpec((D, 1), full),
        pl.BlockSpec((D, D), full),
        pl.BlockSpec((ncat, D), full),
        pl.BlockSpec((1, ncat), full),
    ]
    scratch_shapes = [
        pltpu.VMEM((N, 1), jnp.float32),
        pltpu.VMEM((N, 1), jnp.float32),
        pltpu.VMEM((1, D), jnp.float32),
        pltpu.VMEM((M, DE), jnp.bfloat16),
        pltpu.VMEM((1, M), jnp.float32),
        pltpu.SMEM((1, 1), jnp.float32),
        pltpu.VMEM((N, DE), jnp.bfloat16),
        pltpu.VMEM((1, D), jnp.float32),
        pltpu.VMEM((N, D), jnp.float32),
    ]
    return pl.pallas_call(
        _fwd_kernel,
        grid=grid,
        in_specs=in_specs,
        out_specs=pl.BlockSpec((_BN, ncat), out_map),
        out_shape=jax.ShapeDtypeStruct((N, ncat), jnp.float32),
        scratch_shapes=scratch_shapes,
    )(H, H, x0, W2_1, W3_1, a_1, a2_1, wc_1.reshape(D, 1),
      W2_2, W3_2, a_2, a2_2, wc_2.reshape(D, 1), W_2,
      lin_W, lin_b.reshape(1, ncat))
